# Initial kernel scaffold; baseline (speedup 1.0000x reference)
import jax
import jax.numpy as jnp
from jax.experimental import pallas as pl


def kernel(x, edge_index, batch, W1l, b1, W1r, W2l, b2, W2r, W3l, b3, W3r, Wfc, bfc):
    raise NotImplementedError("write your pallas kernel here")


# speedup vs baseline: 3.8647x; 3.8647x over previous
import functools
import jax
import jax.numpy as jnp
from jax import lax
from jax.experimental import pallas as pl
from jax.experimental.pallas import tpu as pltpu
from jax.experimental.pallas import tpu_sc as plsc

N = 10000
E = 320000
D = 128
OUT = 64
G = 64

NW = 32
CHUNK = 80
NCH = 128
EPW = NCH * CHUNK
NACC = 10240
RPS = NACC // 16
CNTR = NACC // 128
GC = 16
NG = NCH // GC
NB = 4

_mesh = plsc.VectorSubcoreMesh(core_axis_name="c", subcore_axis_name="s")


def _scatter_body(y_hbm, src_hbm, dst_hbm, zeros_hbm, part_hbm,
                  src_v, dst_v, rows_v, acc_sh,
                  g0, g1, g2, g3, s0, s1, s2, s3):
    cid = lax.axis_index("c")
    sid = lax.axis_index("s")
    wid = sid * 2 + cid
    gsem = (g0, g1, g2, g3)
    ssem = (s0, s1, s2, s3)

    pltpu.sync_copy(zeros_hbm, acc_sh.at[pl.ds(sid * RPS, RPS)])
    plsc.subcore_barrier()

    def stage(g, buf):
        pltpu.sync_copy(src_hbm.at[wid, pl.ds(g * GC, GC)], src_v.at[buf])
        pltpu.sync_copy(dst_hbm.at[wid, pl.ds(g * GC, GC)], dst_v.at[buf])

    def wait_gather(b):
        pltpu.make_async_copy(
            y_hbm.at[src_v.at[0, 0]], rows_v.at[b], gsem[b]).wait()

    def wait_scatter(b):
        pltpu.make_async_copy(
            rows_v.at[b], acc_sh.at[pl.ds(0, CHUNK)], ssem[b]).wait()

    stage(0, 0)
    pltpu.async_copy(y_hbm.at[src_v.at[0, 0]], rows_v.at[0], g0)
    pltpu.async_copy(y_hbm.at[src_v.at[0, 1]], rows_v.at[1], g1)

    for g in range(NG):
        if g > 0:
            wait_scatter(2)
            wait_scatter(3)
        if g + 1 < NG:
            stage(g + 1, (g + 1) % 2)
        cur = g % 2
        nxt = (g + 1) % 2

        def round_body(r, c, g=g, cur=cur, nxt=nxt):
            for b in range(NB):
                jj = r * NB + b
                wait_gather(b)
                pltpu.async_copy(rows_v.at[b],
                                 acc_sh.at[dst_v.at[cur, jj]], ssem[b],
                                 add=True)
                bt = (b + 2) % NB

                @pl.when(jj >= 2)
                def _():
                    wait_scatter(bt)

                @pl.when(jj < GC - 2)
                def _():
                    pltpu.async_copy(y_hbm.at[src_v.at[cur, jj + 2]],
                                     rows_v.at[bt], gsem[bt])
                if g + 1 < NG:
                    @pl.when(jj >= GC - 2)
                    def _():
                        pltpu.async_copy(
                            y_hbm.at[src_v.at[nxt, jj - (GC - 2)]],
                            rows_v.at[bt], gsem[bt])
            return c

        lax.fori_loop(0, GC // NB, round_body, 0)

    wait_scatter(2)
    wait_scatter(3)
    plsc.subcore_barrier()
    pltpu.sync_copy(acc_sh.at[pl.ds(sid * RPS, RPS)],
                    part_hbm.at[cid, pl.ds(sid * RPS, RPS)])


_sc_scatter = pl.kernel(
    _scatter_body,
    out_type=jax.ShapeDtypeStruct((2, NACC, D), jnp.float32),
    mesh=_mesh,
    scratch_types=[
        pltpu.VMEM((2, GC, CHUNK), jnp.int32),
        pltpu.VMEM((2, GC, CHUNK), jnp.int32),
        pltpu.VMEM((NB, CHUNK, D), jnp.float32),
        pltpu.VMEM_SHARED((NACC, D), jnp.float32),
    ] + [pltpu.SemaphoreType.DMA] * 8,
    compiler_params=pltpu.CompilerParams(needs_layout_passes=False),
)


def _count_body(dst_hbm, zeros_hbm, cnt_hbm,
                dst_v, cnt_v, rowidx_v, cnt_sh, *_):
    cid = lax.axis_index("c")
    sid = lax.axis_index("s")
    wid = sid * 2 + cid

    pltpu.sync_copy(dst_hbm.at[wid], dst_v)
    pltpu.sync_copy(zeros_hbm, cnt_v)

    @pl.when(sid == 0)
    def _():
        pltpu.sync_copy(zeros_hbm, cnt_sh)

    for t in range(CNTR // 16):
        rowidx_v[pl.ds(t * 16, 16)] = lax.iota(jnp.int32, 16) + t * 16
    plsc.subcore_barrier()

    ones16 = jnp.full((16,), 1.0, jnp.float32)

    def chunk_body(j, c):
        for k in range(CHUNK // 16):
            idx16 = dst_v[j, pl.ds(k * 16, 16)]
            r16 = lax.shift_right_logical(idx16, 7)
            c16 = lax.bitwise_and(idx16, 127)
            plsc.addupdate_scatter(cnt_v, [r16, c16], ones16)
        return c

    lax.fori_loop(0, NCH, chunk_body, 0)

    pltpu.sync_copy(cnt_v, cnt_sh.at[rowidx_v], add=True)
    plsc.subcore_barrier()

    @pl.when(sid == 0)
    def _():
        pltpu.sync_copy(cnt_sh, cnt_hbm.at[cid])


_sc_count = pl.kernel(
    _count_body,
    out_type=jax.ShapeDtypeStruct((2, CNTR, 128), jnp.float32),
    mesh=_mesh,
    scratch_types=[
        pltpu.VMEM((NCH, CHUNK), jnp.int32),
        pltpu.VMEM((CNTR, 128), jnp.float32),
        pltpu.VMEM((CNTR,), jnp.int32),
        pltpu.VMEM_SHARED((CNTR, 128), jnp.float32),
    ],
    compiler_params=pltpu.CompilerParams(needs_layout_passes=False),
)


BN = 1000


def _tc_in_body(x_ref, wl_ref, wr_ref, y_ref, z_ref):
    x = x_ref[...]
    y_ref[...] = jnp.dot(x, wl_ref[...], preferred_element_type=jnp.float32)
    z_ref[...] = jnp.dot(x, wr_ref[...], preferred_element_type=jnp.float32)


def _tc_in(x, Wl, Wr):
    return pl.pallas_call(
        _tc_in_body,
        grid=(N // BN,),
        in_specs=[
            pl.BlockSpec((BN, D), lambda i: (i, 0)),
            pl.BlockSpec((D, D), lambda i: (0, 0)),
            pl.BlockSpec((D, D), lambda i: (0, 0)),
        ],
        out_specs=[
            pl.BlockSpec((BN, D), lambda i: (i, 0)),
            pl.BlockSpec((BN, D), lambda i: (i, 0)),
        ],
        out_shape=[
            jax.ShapeDtypeStruct((N, D), jnp.float32),
            jax.ShapeDtypeStruct((N, D), jnp.float32),
        ],
    )(x, Wl, Wr)


def _lrelu(v):
    return jnp.where(v >= 0, v, 0.01 * v)


def _tc_mid_body(p_ref, z_ref, inv_ref, b_ref, wl_ref, wr_ref, y_ref, zo_ref):
    s = p_ref[0] + p_ref[1]
    h = _lrelu(s * inv_ref[...] + z_ref[...] + b_ref[...])
    y_ref[...] = jnp.dot(h, wl_ref[...], preferred_element_type=jnp.float32)
    zo_ref[...] = jnp.dot(h, wr_ref[...], preferred_element_type=jnp.float32)


def _tc_mid(parts, z, inv, b, Wl, Wr):
    return pl.pallas_call(
        _tc_mid_body,
        grid=(N // BN,),
        in_specs=[
            pl.BlockSpec((2, BN, D), lambda i: (0, i, 0)),
            pl.BlockSpec((BN, D), lambda i: (i, 0)),
            pl.BlockSpec((BN, 1), lambda i: (i, 0)),
            pl.BlockSpec((1, D), lambda i: (0, 0)),
            pl.BlockSpec((D, D), lambda i: (0, 0)),
            pl.BlockSpec((D, D), lambda i: (0, 0)),
        ],
        out_specs=[
            pl.BlockSpec((BN, D), lambda i: (i, 0)),
            pl.BlockSpec((BN, D), lambda i: (i, 0)),
        ],
        out_shape=[
            jax.ShapeDtypeStruct((N, D), jnp.float32),
            jax.ShapeDtypeStruct((N, D), jnp.float32),
        ],
    )(parts, z, inv, b.reshape(1, D), Wl, Wr)


BP = 400


def _tc_out_body(p_ref, z_ref, inv_ref, b_ref, ids_ref, wfc_ref, bfc_ref,
                 o_ref, pool_ref):
    i = pl.program_id(0)

    @pl.when(i == 0)
    def _():
        pool_ref[...] = jnp.full((G, D), -jnp.inf, jnp.float32)

    s = p_ref[0] + p_ref[1]
    h = _lrelu(s * inv_ref[...] + z_ref[...] + b_ref[...])
    ids = ids_ref[...]
    gids = lax.broadcasted_iota(jnp.int32, (G, 1, 1), 0)
    mask = ids[None, :, :] == gids
    hm = jnp.where(mask, h[None, :, :], -jnp.inf)
    pool_ref[...] = jnp.maximum(pool_ref[...], jnp.max(hm, axis=1))

    @pl.when(i == N // BP - 1)
    def _():
        o_ref[...] = (
            jnp.dot(pool_ref[...], wfc_ref[...],
                    preferred_element_type=jnp.float32) + bfc_ref[...])


def _tc_out(parts, z, inv, b, ids, Wfc, bfc):
    return pl.pallas_call(
        _tc_out_body,
        grid=(N // BP,),
        in_specs=[
            pl.BlockSpec((2, BP, D), lambda i: (0, i, 0)),
            pl.BlockSpec((BP, D), lambda i: (i, 0)),
            pl.BlockSpec((BP, 1), lambda i: (i, 0)),
            pl.BlockSpec((1, D), lambda i: (0, 0)),
            pl.BlockSpec((BP, 1), lambda i: (i, 0)),
            pl.BlockSpec((D, OUT), lambda i: (0, 0)),
            pl.BlockSpec((1, OUT), lambda i: (0, 0)),
        ],
        out_specs=pl.BlockSpec((G, OUT), lambda i: (0, 0)),
        out_shape=jax.ShapeDtypeStruct((G, OUT), jnp.float32),
        scratch_shapes=[pltpu.VMEM((G, D), jnp.float32)],
    )(parts, z, inv, b.reshape(1, D), ids, Wfc, bfc.reshape(1, OUT))


def kernel(x, edge_index, batch, W1l, b1, W1r, W2l, b2, W2r, W3l, b3, W3r,
           Wfc, bfc):
    src = edge_index[0].reshape(NW, E // NW)
    dst = edge_index[1].reshape(NW, E // NW)
    pad = EPW - E // NW
    src_p = jnp.concatenate(
        [src, jnp.zeros((NW, pad), jnp.int32)], axis=1).reshape(NW, NCH, CHUNK)
    dst_p = jnp.concatenate(
        [dst, jnp.full((NW, pad), NACC - 1, jnp.int32)],
        axis=1).reshape(NW, NCH, CHUNK)
    zeros = jnp.zeros((RPS, 128), jnp.float32)
    zeros_c = jnp.zeros((CNTR, 128), jnp.float32)
    ids = batch.reshape(N, 1)

    cnts = _sc_count(dst_p, zeros_c)
    cnt = (cnts[0] + cnts[1]).reshape(-1)[:N]
    inv = (1.0 / jnp.clip(cnt, 1.0, None)).reshape(N, 1)

    y1, z1 = _tc_in(x, W1l, W1r)
    parts1 = _sc_scatter(y1, src_p, dst_p, zeros)

    y2, z2 = _tc_mid(parts1, z1, inv, b1, W2l, W2r)
    parts2 = _sc_scatter(y2, src_p, dst_p, zeros)

    y3, z3 = _tc_mid(parts2, z2, inv, b2, W3l, W3r)
    parts3 = _sc_scatter(y3, src_p, dst_p, zeros)

    return _tc_out(parts3, z3, inv, b3, ids, Wfc, bfc)


# speedup vs baseline: 3.9006x; 1.0093x over previous
import functools
import jax
import jax.numpy as jnp
from jax import lax
from jax.experimental import pallas as pl
from jax.experimental.pallas import tpu as pltpu
from jax.experimental.pallas import tpu_sc as plsc

N = 10000
E = 320000
D = 128
OUT = 64
G = 64

NW = 32
CHUNK = 128
NCH = 80
EPW = NCH * CHUNK
NACC = 10240
RPS = NACC // 16
CNTR = NACC // 128
GC = 16

_mesh = plsc.VectorSubcoreMesh(core_axis_name="c", subcore_axis_name="s")


def _sc_body(with_counts, refs):
    if with_counts:
        (y_hbm, src_hbm, dst_hbm, zeros_hbm, part_hbm, cnt_hbm,
         src_v, dst_v, rows_v, cnt_v, rowidx_v,
         acc_sh, cnt_sh, sem0, sem1) = refs
    else:
        (y_hbm, src_hbm, dst_hbm, zeros_hbm, part_hbm,
         src_v, dst_v, rows_v,
         acc_sh, sem0, sem1) = refs

    cid = lax.axis_index("c")
    sid = lax.axis_index("s")
    wid = sid * 2 + cid

    pltpu.sync_copy(zeros_hbm, acc_sh.at[pl.ds(sid * RPS, RPS)])
    if with_counts:
        pltpu.sync_copy(zeros_hbm.at[pl.ds(0, CNTR)], cnt_v)

        @pl.when(sid == 0)
        def _():
            pltpu.sync_copy(zeros_hbm.at[pl.ds(0, CNTR)], cnt_sh)

        for t in range(CNTR // 16):
            rowidx_v[pl.ds(t * 16, 16)] = lax.iota(jnp.int32, 16) + t * 16
    plsc.subcore_barrier()

    ones16 = jnp.full((16,), 1.0, jnp.float32)

    def loop_grp(grp, carry):
        pltpu.sync_copy(src_hbm.at[wid, pl.ds(grp * GC, GC)], src_v)
        pltpu.sync_copy(dst_hbm.at[wid, pl.ds(grp * GC, GC)], dst_v)
        pltpu.async_copy(y_hbm.at[src_v.at[0]], rows_v.at[0], sem0)
        pltpu.async_copy(y_hbm.at[src_v.at[1]], rows_v.at[1], sem1)

        def pair_body(i, c):
            for b in range(2):
                j = i * 2 + b
                sem = sem0 if b == 0 else sem1
                pltpu.make_async_copy(
                    y_hbm.at[src_v.at[0]], rows_v.at[b], sem).wait()
                if with_counts:
                    for k in range(8):
                        idx16 = dst_v[j, pl.ds(k * 16, 16)]
                        r16 = lax.shift_right_logical(idx16, 7)
                        c16 = lax.bitwise_and(idx16, 127)
                        plsc.addupdate_scatter(cnt_v, [r16, c16], ones16)
                pltpu.sync_copy(rows_v.at[b], acc_sh.at[dst_v.at[j]], add=True)

                @pl.when(j + 2 < GC)
                def _():
                    pltpu.async_copy(
                        y_hbm.at[src_v.at[j + 2]], rows_v.at[b], sem)
            return c

        lax.fori_loop(0, GC // 2, pair_body, 0)
        return carry

    lax.fori_loop(0, NCH // GC, loop_grp, 0)

    if with_counts:
        pltpu.sync_copy(cnt_v, cnt_sh.at[rowidx_v], add=True)
    plsc.subcore_barrier()

    pltpu.sync_copy(acc_sh.at[pl.ds(sid * RPS, RPS)],
                    part_hbm.at[cid, pl.ds(sid * RPS, RPS)])
    if with_counts:
        @pl.when(sid == 0)
        def _():
            pltpu.sync_copy(cnt_sh, cnt_hbm.at[cid])


def _make_sc_kernel(with_counts):
    out_type = [jax.ShapeDtypeStruct((2, NACC, D), jnp.float32)]
    if with_counts:
        out_type.append(jax.ShapeDtypeStruct((2, CNTR, 128), jnp.float32))
    scratch = [
        pltpu.VMEM((GC, CHUNK), jnp.int32),
        pltpu.VMEM((GC, CHUNK), jnp.int32),
        pltpu.VMEM((2, CHUNK, D), jnp.float32),
    ]
    if with_counts:
        scratch += [
            pltpu.VMEM((CNTR, 128), jnp.float32),
            pltpu.VMEM((CNTR,), jnp.int32),
        ]
    scratch_sh = [pltpu.VMEM_SHARED((NACC, D), jnp.float32)]
    if with_counts:
        scratch_sh.append(pltpu.VMEM_SHARED((CNTR, 128), jnp.float32))
    scratch += scratch_sh + [pltpu.SemaphoreType.DMA, pltpu.SemaphoreType.DMA]

    def wrapped(*refs):
        _sc_body(with_counts, refs)

    return pl.kernel(
        wrapped,
        out_type=tuple(out_type) if with_counts else out_type[0],
        mesh=_mesh,
        scratch_types=scratch,
        compiler_params=pltpu.CompilerParams(needs_layout_passes=False),
    )


_sc_scatter_counts = _make_sc_kernel(True)
_sc_scatter = _make_sc_kernel(False)


BN = 1000


def _tc_in_body(x_ref, wl_ref, wr_ref, y_ref, z_ref):
    x = x_ref[...]
    y_ref[...] = jnp.dot(x, wl_ref[...], preferred_element_type=jnp.float32)
    z_ref[...] = jnp.dot(x, wr_ref[...], preferred_element_type=jnp.float32)


def _tc_in(x, Wl, Wr):
    return pl.pallas_call(
        _tc_in_body,
        grid=(N // BN,),
        in_specs=[
            pl.BlockSpec((BN, D), lambda i: (i, 0)),
            pl.BlockSpec((D, D), lambda i: (0, 0)),
            pl.BlockSpec((D, D), lambda i: (0, 0)),
        ],
        out_specs=[
            pl.BlockSpec((BN, D), lambda i: (i, 0)),
            pl.BlockSpec((BN, D), lambda i: (i, 0)),
        ],
        out_shape=[
            jax.ShapeDtypeStruct((N, D), jnp.float32),
            jax.ShapeDtypeStruct((N, D), jnp.float32),
        ],
    )(x, Wl, Wr)


def _lrelu(v):
    return jnp.where(v >= 0, v, 0.01 * v)


def _tc_mid_body(p_ref, z_ref, inv_ref, b_ref, wl_ref, wr_ref, y_ref, zo_ref):
    s = p_ref[0] + p_ref[1]
    h = _lrelu(s * inv_ref[...] + z_ref[...] + b_ref[...])
    y_ref[...] = jnp.dot(h, wl_ref[...], preferred_element_type=jnp.float32)
    zo_ref[...] = jnp.dot(h, wr_ref[...], preferred_element_type=jnp.float32)


def _tc_mid(parts, z, inv, b, Wl, Wr):
    return pl.pallas_call(
        _tc_mid_body,
        grid=(N // BN,),
        in_specs=[
            pl.BlockSpec((2, BN, D), lambda i: (0, i, 0)),
            pl.BlockSpec((BN, D), lambda i: (i, 0)),
            pl.BlockSpec((BN, 1), lambda i: (i, 0)),
            pl.BlockSpec((1, D), lambda i: (0, 0)),
            pl.BlockSpec((D, D), lambda i: (0, 0)),
            pl.BlockSpec((D, D), lambda i: (0, 0)),
        ],
        out_specs=[
            pl.BlockSpec((BN, D), lambda i: (i, 0)),
            pl.BlockSpec((BN, D), lambda i: (i, 0)),
        ],
        out_shape=[
            jax.ShapeDtypeStruct((N, D), jnp.float32),
            jax.ShapeDtypeStruct((N, D), jnp.float32),
        ],
    )(parts, z, inv, b.reshape(1, D), Wl, Wr)


BP = 400


def _tc_out_body(p_ref, z_ref, inv_ref, b_ref, ids_ref, wfc_ref, bfc_ref,
                 o_ref, pool_ref):
    i = pl.program_id(0)

    @pl.when(i == 0)
    def _():
        pool_ref[...] = jnp.full((G, D), -jnp.inf, jnp.float32)

    s = p_ref[0] + p_ref[1]
    h = _lrelu(s * inv_ref[...] + z_ref[...] + b_ref[...])
    ids = ids_ref[...]
    gids = lax.broadcasted_iota(jnp.int32, (G, 1, 1), 0)
    mask = ids[None, :, :] == gids
    hm = jnp.where(mask, h[None, :, :], -jnp.inf)
    pool_ref[...] = jnp.maximum(pool_ref[...], jnp.max(hm, axis=1))

    @pl.when(i == N // BP - 1)
    def _():
        o_ref[...] = (
            jnp.dot(pool_ref[...], wfc_ref[...],
                    preferred_element_type=jnp.float32) + bfc_ref[...])


def _tc_out(parts, z, inv, b, ids, Wfc, bfc):
    return pl.pallas_call(
        _tc_out_body,
        grid=(N // BP,),
        in_specs=[
            pl.BlockSpec((2, BP, D), lambda i: (0, i, 0)),
            pl.BlockSpec((BP, D), lambda i: (i, 0)),
            pl.BlockSpec((BP, 1), lambda i: (i, 0)),
            pl.BlockSpec((1, D), lambda i: (0, 0)),
            pl.BlockSpec((BP, 1), lambda i: (i, 0)),
            pl.BlockSpec((D, OUT), lambda i: (0, 0)),
            pl.BlockSpec((1, OUT), lambda i: (0, 0)),
        ],
        out_specs=pl.BlockSpec((G, OUT), lambda i: (0, 0)),
        out_shape=jax.ShapeDtypeStruct((G, OUT), jnp.float32),
        scratch_shapes=[pltpu.VMEM((G, D), jnp.float32)],
    )(parts, z, inv, b.reshape(1, D), ids, Wfc, bfc.reshape(1, OUT))


def kernel(x, edge_index, batch, W1l, b1, W1r, W2l, b2, W2r, W3l, b3, W3r,
           Wfc, bfc):
    src = edge_index[0].reshape(NW, E // NW)
    dst = edge_index[1].reshape(NW, E // NW)
    pad = EPW - E // NW
    src_p = jnp.concatenate(
        [src, jnp.zeros((NW, pad), jnp.int32)], axis=1).reshape(NW, NCH, CHUNK)
    dst_p = jnp.concatenate(
        [dst, jnp.full((NW, pad), NACC - 1, jnp.int32)],
        axis=1).reshape(NW, NCH, CHUNK)
    zeros = jnp.zeros((RPS, 128), jnp.float32)
    ids = batch.reshape(N, 1)

    y1, z1 = _tc_in(x, W1l, W1r)
    parts1, cnts = _sc_scatter_counts(y1, src_p, dst_p, zeros)
    cnt = (cnts[0] + cnts[1]).reshape(-1)[:N]
    inv = (1.0 / jnp.clip(cnt, 1.0, None)).reshape(N, 1)

    y2, z2 = _tc_mid(parts1, z1, inv, b1, W2l, W2r)
    parts2 = _sc_scatter(y2, src_p, dst_p, zeros)

    y3, z3 = _tc_mid(parts2, z2, inv, b2, W3l, W3r)
    parts3 = _sc_scatter(y3, src_p, dst_p, zeros)

    return _tc_out(parts3, z3, inv, b3, ids, Wfc, bfc)
